# baseline (speedup 1.0000x reference)
import functools

import jax
import jax.numpy as jnp
from jax import lax
from jax.experimental import pallas as pl
from jax.experimental.pallas import tpu as pltpu
from jax.experimental.pallas import tpu_sc as plsc

_E = 64
_D = 1024
_F = 256
_T = 2048
_CAP = 256

_NC = 2
_NS = 16
_NW = _NC * _NS
_L = 16
_SENT = _E * _CAP


def _router_body(x_ref, gwt_ref, bias_ref, d1_ref, d2_ref, s1_ref, s2_ref,
                 w1_ref, w2_ref):
    x = x_ref[...]
    logits = jnp.dot(x, gwt_ref[...], preferred_element_type=jnp.float32)
    scores = jax.nn.sigmoid(logits)
    choice = scores + bias_ref[...]
    lane = jax.lax.broadcasted_iota(jnp.int32, (_T, _E), 1)
    neg = jnp.float32(-jnp.inf)

    m1 = jnp.max(choice, axis=1, keepdims=True)
    i1 = jnp.min(jnp.where(choice == m1, lane, _E), axis=1, keepdims=True)
    sel1 = lane == i1
    s1 = jnp.max(jnp.where(sel1, scores, neg), axis=1, keepdims=True)

    choice2 = jnp.where(sel1, neg, choice)
    m2 = jnp.max(choice2, axis=1, keepdims=True)
    i2 = jnp.min(jnp.where(choice2 == m2, lane, _E), axis=1, keepdims=True)
    sel2 = lane == i2
    s2 = jnp.max(jnp.where(sel2, scores, neg), axis=1, keepdims=True)

    denom = s1 + s2 + jnp.float32(1e-20)

    onehot = sel1.astype(jnp.float32) + sel2.astype(jnp.float32)
    incl = onehot
    step = 1
    while step < _T:
        shifted = jnp.pad(incl, ((step, 0), (0, 0)))[:_T]
        incl = incl + shifted
        step *= 2
    excl = incl - onehot
    pos1 = jnp.sum(jnp.where(sel1, excl, 0.0), axis=1, keepdims=True)
    pos2 = jnp.sum(jnp.where(sel2, excl, 0.0), axis=1, keepdims=True)
    pos1 = pos1.astype(jnp.int32)
    pos2 = pos2.astype(jnp.int32)
    valid1 = pos1 < _CAP
    valid2 = pos2 < _CAP

    slot1 = i1 * _CAP + jnp.minimum(pos1, _CAP - 1)
    slot2 = i2 * _CAP + jnp.minimum(pos2, _CAP - 1)
    d1_ref[...] = jnp.where(valid1, slot1, _SENT)
    d2_ref[...] = jnp.where(valid2, slot2, _SENT)
    s1_ref[...] = slot1
    s2_ref[...] = slot2
    w1_ref[...] = jnp.where(valid1, s1 / denom, 0.0)
    w2_ref[...] = jnp.where(valid2, s2 / denom, 0.0)


def _router(x, gate_weight, bias):
    out = pl.pallas_call(
        _router_body,
        out_shape=[
            jax.ShapeDtypeStruct((_T, 1), jnp.int32),
            jax.ShapeDtypeStruct((_T, 1), jnp.int32),
            jax.ShapeDtypeStruct((_T, 1), jnp.int32),
            jax.ShapeDtypeStruct((_T, 1), jnp.int32),
            jax.ShapeDtypeStruct((_T, 1), jnp.float32),
            jax.ShapeDtypeStruct((_T, 1), jnp.float32),
        ],
    )(x, gate_weight.T, bias.reshape(1, _E))
    return [o.reshape(_T) for o in out]


def _expert_body(xs_ref, w1_ref, w3_ref, w2_ref, y_ref):
    xs = xs_ref[0]
    a = jnp.dot(xs, w1_ref[0], preferred_element_type=jnp.float32)
    b = jnp.dot(xs, w3_ref[0], preferred_element_type=jnp.float32)
    h = a * jax.nn.sigmoid(a) * b
    y_ref[0] = jnp.dot(h, w2_ref[0], preferred_element_type=jnp.float32)


def _experts(xs, w1, w3, w2):
    return pl.pallas_call(
        _expert_body,
        grid=(_E,),
        in_specs=[
            pl.BlockSpec((1, _CAP, _D), lambda e: (e, 0, 0)),
            pl.BlockSpec((1, _D, _F), lambda e: (e, 0, 0)),
            pl.BlockSpec((1, _D, _F), lambda e: (e, 0, 0)),
            pl.BlockSpec((1, _F, _D), lambda e: (e, 0, 0)),
        ],
        out_specs=pl.BlockSpec((1, _CAP, _D), lambda e: (e, 0, 0)),
        out_shape=jax.ShapeDtypeStruct((_E, _CAP, _D), jnp.float32),
    )(xs, w1, w3, w2)


_EPW = _E // _NW
_SPW = _EPW * _CAP
_GCH = 32


def _dispatch_body(d1_hbm, d2_hbm, x_hbm, xs_hbm, s1v, s2v, toks,
                   rowbuf, rowbuf2, sem, sem2, osem, osem2):
    wid = lax.axis_index("s") * _NC + lax.axis_index("c")
    pltpu.sync_copy(d1_hbm, s1v)
    pltpu.sync_copy(d2_hbm, s2v)

    zeros = jnp.zeros((_L,), jnp.int32)

    def zbody(i, carry):
        toks[pl.ds(i * _L, _L)] = zeros
        return carry

    lax.fori_loop(0, _SPW // _L, zbody, 0)

    lanes = lax.broadcasted_iota(jnp.int32, (_L,), 0)
    lo = jnp.full((_L,), wid * _SPW, jnp.int32)
    span = jnp.full((_L,), _SPW, jnp.int32)
    dump = span + lanes

    def sbody(j, carry):
        tv = lanes + jnp.full((_L,), j * _L, jnp.int32)
        for sv in (s1v, s2v):
            rel = sv[pl.ds(j * _L, _L)] - lo
            ok = (rel >= zeros) & (rel < span)
            plsc.store_scatter(toks, [jnp.where(ok, rel, dump)], tv)
        return carry

    lax.fori_loop(0, _T // _L, sbody, 0)

    nch = _SPW // _GCH
    bufs = (rowbuf, rowbuf2)
    gsems = (sem, sem2)
    osems = (osem, osem2)
    gets = [None] * nch
    puts = [None] * nch
    for c in range(nch):
        b = c % 2
        if c >= 2:
            puts[c - 2].wait()
        gets[c] = pltpu.async_copy(
            x_hbm.at[toks.at[pl.ds(c * _GCH, _GCH)]], bufs[b], gsems[b])
        if c >= 1:
            gets[c - 1].wait()
            puts[c - 1] = pltpu.async_copy(
                bufs[1 - b],
                xs_hbm.at[pl.ds(wid * _SPW + (c - 1) * _GCH, _GCH)],
                osems[1 - b])
    gets[nch - 1].wait()
    puts[nch - 1] = pltpu.async_copy(
        bufs[(nch - 1) % 2],
        xs_hbm.at[pl.ds(wid * _SPW + (nch - 1) * _GCH, _GCH)],
        osems[(nch - 1) % 2])
    puts[nch - 2].wait()
    puts[nch - 1].wait()


def _dispatch(x, d1, d2):
    f = functools.partial(
        pl.kernel,
        out_type=jax.ShapeDtypeStruct((_E * _CAP, _D), jnp.float32),
        mesh=plsc.VectorSubcoreMesh(core_axis_name="c", subcore_axis_name="s"),
        compiler_params=pltpu.CompilerParams(needs_layout_passes=False),
        scratch_types=[
            pltpu.VMEM((_T,), jnp.int32),
            pltpu.VMEM((_T,), jnp.int32),
            pltpu.VMEM((_SPW + _L,), jnp.int32),
            pltpu.VMEM((_GCH, _D), jnp.float32),
            pltpu.VMEM((_GCH, _D), jnp.float32),
            pltpu.SemaphoreType.DMA,
            pltpu.SemaphoreType.DMA,
            pltpu.SemaphoreType.DMA,
            pltpu.SemaphoreType.DMA,
        ],
    )(_dispatch_body)
    return f(d1, d2, x)


_TPW = _T // _NW
_CCH = 32


def _combine_body(y_hbm, s1_hbm, s2_hbm, w1_hbm, w2_hbm, out_hbm,
                  s1v, s2v, w1v, w2v, y1buf, y2buf, sem1, sem2):
    wid = lax.axis_index("s") * _NC + lax.axis_index("c")
    for c in range(_TPW // _CCH):
        tb = wid * _TPW + c * _CCH
        pltpu.sync_copy(s1_hbm.at[pl.ds(tb, _CCH)], s1v)
        pltpu.sync_copy(s2_hbm.at[pl.ds(tb, _CCH)], s2v)
        pltpu.sync_copy(w1_hbm.at[pl.ds(tb, _CCH)], w1v.at[pl.ds(0, _CCH)])
        pltpu.sync_copy(w2_hbm.at[pl.ds(tb, _CCH)], w2v.at[pl.ds(0, _CCH)])
        cp1 = pltpu.async_copy(y_hbm.at[s1v], y1buf, sem1)
        cp2 = pltpu.async_copy(y_hbm.at[s2v], y2buf, sem2)
        cp1.wait()
        cp2.wait()

        def tbody(t, carry):
            a = jnp.broadcast_to(w1v[pl.ds(t, _L)][0], (_L,))
            b = jnp.broadcast_to(w2v[pl.ds(t, _L)][0], (_L,))

            def vbody(v, carry2):
                y1buf[t, pl.ds(v * _L, _L)] = (
                    y1buf[t, pl.ds(v * _L, _L)] * a
                    + y2buf[t, pl.ds(v * _L, _L)] * b)
                return carry2

            return lax.fori_loop(0, _D // _L, vbody, carry)

        lax.fori_loop(0, _CCH, tbody, 0)
        pltpu.sync_copy(y1buf, out_hbm.at[pl.ds(tb, _CCH)])


def _combine(y, slot1, slot2, cw1, cw2):
    f = functools.partial(
        pl.kernel,
        out_type=jax.ShapeDtypeStruct((_T, _D), jnp.float32),
        mesh=plsc.VectorSubcoreMesh(core_axis_name="c", subcore_axis_name="s"),
        compiler_params=pltpu.CompilerParams(needs_layout_passes=False),
        scratch_types=[
            pltpu.VMEM((_CCH,), jnp.int32),
            pltpu.VMEM((_CCH,), jnp.int32),
            pltpu.VMEM((_CCH + _L,), jnp.float32),
            pltpu.VMEM((_CCH + _L,), jnp.float32),
            pltpu.VMEM((_CCH, _D), jnp.float32),
            pltpu.VMEM((_CCH, _D), jnp.float32),
            pltpu.SemaphoreType.DMA,
            pltpu.SemaphoreType.DMA,
        ],
    )(_combine_body)
    return f(y, slot1, slot2, cw1, cw2)


def kernel(hidden_states, gate_weight, w1, w3, w2, e_score_correction_bias):
    x = hidden_states.reshape(_T, _D)
    d1, d2, slot1, slot2, cw1, cw2 = _router(
        x, gate_weight, e_score_correction_bias)
    xs = _dispatch(x, d1, d2).reshape(_E, _CAP, _D)
    y = _experts(xs, w1, w3, w2).reshape(_E * _CAP, _D)
    out = _combine(y, slot1, slot2, cw1, cw2)
    return out.reshape(hidden_states.shape)


# speedup vs baseline: 2.9353x; 2.9353x over previous
import functools

import jax
import jax.numpy as jnp
from jax import lax
from jax.experimental import pallas as pl
from jax.experimental.pallas import tpu as pltpu
from jax.experimental.pallas import tpu_sc as plsc

_E = 64
_D = 1024
_F = 256
_T = 2048
_CAP = 256

_NC = 2
_NS = 16
_NW = _NC * _NS
_L = 16
_SENT = _E * _CAP


def _router_body(x_ref, gwt_ref, bias_ref, d1_ref, d2_ref, s1_ref, s2_ref,
                 w1_ref, w2_ref, cnt_ref):
    x = x_ref[...]
    logits = jnp.dot(x, gwt_ref[...], preferred_element_type=jnp.float32)
    scores = jax.nn.sigmoid(logits)
    choice = scores + bias_ref[...]
    lane = jax.lax.broadcasted_iota(jnp.int32, (_T, _E), 1)
    neg = jnp.float32(-jnp.inf)

    m1 = jnp.max(choice, axis=1, keepdims=True)
    i1 = jnp.min(jnp.where(choice == m1, lane, _E), axis=1, keepdims=True)
    sel1 = lane == i1
    s1 = jnp.max(jnp.where(sel1, scores, neg), axis=1, keepdims=True)

    choice2 = jnp.where(sel1, neg, choice)
    m2 = jnp.max(choice2, axis=1, keepdims=True)
    i2 = jnp.min(jnp.where(choice2 == m2, lane, _E), axis=1, keepdims=True)
    sel2 = lane == i2
    s2 = jnp.max(jnp.where(sel2, scores, neg), axis=1, keepdims=True)

    denom = s1 + s2 + jnp.float32(1e-20)

    onehot = sel1.astype(jnp.float32) + sel2.astype(jnp.float32)
    incl = onehot
    step = 1
    while step < _T:
        shifted = jnp.pad(incl, ((step, 0), (0, 0)))[:_T]
        incl = incl + shifted
        step *= 2
    excl = incl - onehot
    pos1 = jnp.sum(jnp.where(sel1, excl, 0.0), axis=1, keepdims=True)
    pos2 = jnp.sum(jnp.where(sel2, excl, 0.0), axis=1, keepdims=True)
    pos1 = pos1.astype(jnp.int32)
    pos2 = pos2.astype(jnp.int32)
    valid1 = pos1 < _CAP
    valid2 = pos2 < _CAP

    slot1 = i1 * _CAP + jnp.minimum(pos1, _CAP - 1)
    slot2 = i2 * _CAP + jnp.minimum(pos2, _CAP - 1)
    d1_ref[...] = jnp.where(valid1, slot1, _SENT)
    d2_ref[...] = jnp.where(valid2, slot2, _SENT)
    s1_ref[...] = slot1
    s2_ref[...] = slot2
    w1_ref[...] = jnp.where(valid1, s1 / denom, 0.0)
    w2_ref[...] = jnp.where(valid2, s2 / denom, 0.0)
    cnt_ref[...] = incl[_T - 1:_T, :].astype(jnp.int32)


def _router(x, gate_weight, bias):
    out = pl.pallas_call(
        _router_body,
        out_shape=[
            jax.ShapeDtypeStruct((_T, 1), jnp.int32),
            jax.ShapeDtypeStruct((_T, 1), jnp.int32),
            jax.ShapeDtypeStruct((_T, 1), jnp.int32),
            jax.ShapeDtypeStruct((_T, 1), jnp.int32),
            jax.ShapeDtypeStruct((_T, 1), jnp.float32),
            jax.ShapeDtypeStruct((_T, 1), jnp.float32),
            jax.ShapeDtypeStruct((1, _E), jnp.int32),
        ],
    )(x, gate_weight.T, bias.reshape(1, _E))
    return [o.reshape(_T) for o in out[:6]] + [out[6].reshape(_E)]


def _expert_body(xs_ref, w1_ref, w3_ref, w2_ref, y_ref):
    xs = xs_ref[0]
    a = jnp.dot(xs, w1_ref[0], preferred_element_type=jnp.float32)
    b = jnp.dot(xs, w3_ref[0], preferred_element_type=jnp.float32)
    h = a * jax.nn.sigmoid(a) * b
    y_ref[0] = jnp.dot(h, w2_ref[0], preferred_element_type=jnp.float32)


def _experts(xs, w1, w3, w2):
    return pl.pallas_call(
        _expert_body,
        grid=(_E,),
        in_specs=[
            pl.BlockSpec((1, _CAP, _D), lambda e: (e, 0, 0)),
            pl.BlockSpec((1, _D, _F), lambda e: (e, 0, 0)),
            pl.BlockSpec((1, _D, _F), lambda e: (e, 0, 0)),
            pl.BlockSpec((1, _F, _D), lambda e: (e, 0, 0)),
        ],
        out_specs=pl.BlockSpec((1, _CAP, _D), lambda e: (e, 0, 0)),
        out_shape=jax.ShapeDtypeStruct((_E, _CAP, _D), jnp.float32),
    )(xs, w1, w3, w2)


_EPW = _E // _NW
_SPW = _EPW * _CAP
_GCH = 32


def _dispatch_body(d1_hbm, d2_hbm, cnt_hbm, x_hbm, xs_hbm, s1v, s2v, toks,
                   cntv, rowbuf, rowbuf2, sem, sem2, osem, osem2):
    wid = lax.axis_index("s") * _NC + lax.axis_index("c")
    pltpu.sync_copy(d1_hbm, s1v)
    pltpu.sync_copy(d2_hbm, s2v)
    pltpu.sync_copy(cnt_hbm, cntv.at[pl.ds(0, _E)])

    zeros = jnp.zeros((_L,), jnp.int32)

    def zbody(i, carry):
        toks[pl.ds(i * _L, _L)] = zeros
        return carry

    lax.fori_loop(0, _SPW // _L, zbody, 0)

    lanes = lax.broadcasted_iota(jnp.int32, (_L,), 0)
    lo = jnp.full((_L,), wid * _SPW, jnp.int32)
    span = jnp.full((_L,), _SPW, jnp.int32)
    dump = span + lanes

    def sbody(j, carry):
        tv = lanes + jnp.full((_L,), j * _L, jnp.int32)
        for sv in (s1v, s2v):
            rel = sv[pl.ds(j * _L, _L)] - lo
            ok = (rel >= zeros) & (rel < span)
            plsc.store_scatter(toks, [jnp.where(ok, rel, dump)], tv)
        return carry

    lax.fori_loop(0, _T // _L, sbody, 0)

    cv = cntv[pl.ds(_EPW * wid, _L)]
    for sub in range(_EPW):
        nch = (jnp.minimum(cv[sub], _CAP) + (_GCH - 1)) // _GCH
        ebase = (wid * _EPW + sub) * _CAP
        tbase = sub * _CAP

        def gbody(ci, carry, ebase=ebase, tbase=tbase):
            pltpu.async_copy(
                x_hbm.at[toks.at[pl.ds(tbase + ci * _GCH, _GCH)]],
                rowbuf, sem).wait()
            pltpu.sync_copy(
                rowbuf, xs_hbm.at[pl.ds(ebase + ci * _GCH, _GCH)])
            return carry

        lax.fori_loop(0, nch, gbody, 0)


def _dispatch(x, d1, d2, cnt):
    f = functools.partial(
        pl.kernel,
        out_type=jax.ShapeDtypeStruct((_E * _CAP, _D), jnp.float32),
        mesh=plsc.VectorSubcoreMesh(core_axis_name="c", subcore_axis_name="s"),
        compiler_params=pltpu.CompilerParams(needs_layout_passes=False),
        scratch_types=[
            pltpu.VMEM((_T,), jnp.int32),
            pltpu.VMEM((_T,), jnp.int32),
            pltpu.VMEM((_SPW + _L,), jnp.int32),
            pltpu.VMEM((_E + _L,), jnp.int32),
            pltpu.VMEM((_GCH, _D), jnp.float32),
            pltpu.VMEM((_GCH, _D), jnp.float32),
            pltpu.SemaphoreType.DMA,
            pltpu.SemaphoreType.DMA,
            pltpu.SemaphoreType.DMA,
            pltpu.SemaphoreType.DMA,
        ],
    )(_dispatch_body)
    return f(d1, d2, cnt, x)


_TPW = _T // _NW
_CCH = 32


def _combine_body(y_hbm, s1_hbm, s2_hbm, w1_hbm, w2_hbm, out_hbm,
                  s1v, s2v, w1v, w2v, y1buf, y2buf, sem1, sem2):
    wid = lax.axis_index("s") * _NC + lax.axis_index("c")
    for c in range(_TPW // _CCH):
        tb = wid * _TPW + c * _CCH
        pltpu.sync_copy(s1_hbm.at[pl.ds(tb, _CCH)], s1v)
        pltpu.sync_copy(s2_hbm.at[pl.ds(tb, _CCH)], s2v)
        pltpu.sync_copy(w1_hbm.at[pl.ds(tb, _CCH)], w1v.at[pl.ds(0, _CCH)])
        pltpu.sync_copy(w2_hbm.at[pl.ds(tb, _CCH)], w2v.at[pl.ds(0, _CCH)])
        cp1 = pltpu.async_copy(y_hbm.at[s1v], y1buf, sem1)
        cp2 = pltpu.async_copy(y_hbm.at[s2v], y2buf, sem2)
        cp1.wait()
        cp2.wait()

        def tbody(t, carry):
            a = jnp.broadcast_to(w1v[pl.ds(t, _L)][0], (_L,))
            b = jnp.broadcast_to(w2v[pl.ds(t, _L)][0], (_L,))

            def vbody(v, carry2):
                y1buf[t, pl.ds(v * _L, _L)] = (
                    y1buf[t, pl.ds(v * _L, _L)] * a
                    + y2buf[t, pl.ds(v * _L, _L)] * b)
                return carry2

            return lax.fori_loop(0, _D // _L, vbody, carry)

        lax.fori_loop(0, _CCH, tbody, 0)
        pltpu.sync_copy(y1buf, out_hbm.at[pl.ds(tb, _CCH)])


def _combine(y, slot1, slot2, cw1, cw2):
    f = functools.partial(
        pl.kernel,
        out_type=jax.ShapeDtypeStruct((_T, _D), jnp.float32),
        mesh=plsc.VectorSubcoreMesh(core_axis_name="c", subcore_axis_name="s"),
        compiler_params=pltpu.CompilerParams(needs_layout_passes=False),
        scratch_types=[
            pltpu.VMEM((_CCH,), jnp.int32),
            pltpu.VMEM((_CCH,), jnp.int32),
            pltpu.VMEM((_CCH + _L,), jnp.float32),
            pltpu.VMEM((_CCH + _L,), jnp.float32),
            pltpu.VMEM((_CCH, _D), jnp.float32),
            pltpu.VMEM((_CCH, _D), jnp.float32),
            pltpu.SemaphoreType.DMA,
            pltpu.SemaphoreType.DMA,
        ],
    )(_combine_body)
    return f(y, slot1, slot2, cw1, cw2)


def kernel(hidden_states, gate_weight, w1, w3, w2, e_score_correction_bias):
    x = hidden_states.reshape(_T, _D)
    d1, d2, slot1, slot2, cw1, cw2, cnt = _router(
        x, gate_weight, e_score_correction_bias)
    xs = _dispatch(x, d1, d2, cnt).reshape(_E, _CAP, _D)
    y = _experts(xs, w1, w3, w2).reshape(_E * _CAP, _D)
    out = _combine(y, slot1, slot2, cw1, cw2)
    return out.reshape(hidden_states.shape)
